# baseline (speedup 1.0000x reference)
import functools

import jax
import jax.numpy as jnp
from jax import lax
from jax.experimental import pallas as pl
from jax.experimental.pallas import tpu as pltpu
from jax.experimental.pallas import tpu_sc as plsc

NC = 2
NS = 16
NW = NC * NS
LANES = 16
GB = 8192
CAP = GB + 96
FB = 64


def _sc_mesh():
    return plsc.VectorSubcoreMesh(
        core_axis_name="c", subcore_axis_name="s",
        num_cores=NC, num_subcores=NS)


def _iota16():
    return lax.iota(jnp.int32, LANES)


def _make_slots_kernel(EPT, SLOTS, R):
    NIT = EPT // LANES

    @functools.partial(
        pl.kernel,
        out_type=(jax.ShapeDtypeStruct((NW, SLOTS), jnp.float32),
                  jax.ShapeDtypeStruct((NW * EPT,), jnp.int32)),
        mesh=_sc_mesh(),
        compiler_params=pltpu.CompilerParams(needs_layout_passes=False),
        scratch_types=[
            pltpu.VMEM((EPT,), jnp.int32),
            pltpu.VMEM((EPT,), jnp.int32),
            pltpu.VMEM((EPT,), jnp.int32),
            pltpu.VMEM((SLOTS,), jnp.float32),
        ],
    )
    def slots_k(dst_hbm, typ_hbm, cnt_out, g_out, dst_v, typ_v, g_v, acc_v):
        cid = lax.axis_index("c")
        sid = lax.axis_index("s")
        w = cid * NS + sid
        pltpu.sync_copy(dst_hbm.at[pl.ds(w * EPT, EPT)], dst_v)
        pltpu.sync_copy(typ_hbm.at[pl.ds(w * EPT, EPT)], typ_v)

        def zero_body(i, carry):
            acc_v[pl.ds(i * LANES, LANES)] = jnp.zeros((LANES,), jnp.float32)
            return carry

        lax.fori_loop(0, SLOTS // LANES, zero_body, 0)
        ones = jnp.ones((LANES,), jnp.float32)

        def body(i, carry):
            d = dst_v[pl.ds(i * LANES, LANES)]
            t = typ_v[pl.ds(i * LANES, LANES)]
            g = d * R + t
            g_v[pl.ds(i * LANES, LANES)] = g
            plsc.addupdate_scatter(acc_v, [g], ones)
            return carry

        lax.fori_loop(0, NIT, body, 0)
        pltpu.sync_copy(g_v, g_out.at[pl.ds(w * EPT, EPT)])
        pltpu.sync_copy(acc_v, cnt_out.at[w])

    return slots_k


def _make_cnt_reduce(SLOTS):
    SL = SLOTS // NW

    @functools.partial(
        pl.kernel,
        out_type=jax.ShapeDtypeStruct((SLOTS,), jnp.float32),
        mesh=_sc_mesh(),
        compiler_params=pltpu.CompilerParams(needs_layout_passes=False),
        scratch_types=[
            pltpu.VMEM((SL,), jnp.float32),
            pltpu.VMEM((SL,), jnp.float32),
        ],
    )
    def reduce_k(parts_hbm, out_hbm, acc_v, buf_v):
        cid = lax.axis_index("c")
        sid = lax.axis_index("s")
        w = cid * NS + sid

        def zero_body(i, carry):
            acc_v[pl.ds(i * LANES, LANES)] = jnp.zeros((LANES,), jnp.float32)
            return carry

        lax.fori_loop(0, SL // LANES, zero_body, 0)

        def outer(j, carry):
            pltpu.sync_copy(parts_hbm.at[j, pl.ds(w * SL, SL)], buf_v)

            def inner(i, c2):
                sl = pl.ds(i * LANES, LANES)
                acc_v[sl] = acc_v[sl] + buf_v[sl]
                return c2

            lax.fori_loop(0, SL // LANES, inner, 0)
            return carry

        lax.fori_loop(0, NW, outer, 0)
        pltpu.sync_copy(acc_v, out_hbm.at[pl.ds(w * SL, SL)])

    return reduce_k


def _make_sums(EPAD, D, ACC_ROWS, PASSES):
    NSB = EPAD // GB
    AROWS = ACC_ROWS + 8
    TOT = PASSES * NW * ACC_ROWS

    @functools.partial(
        pl.kernel,
        out_type=jax.ShapeDtypeStruct((TOT, D), jnp.float32),
        mesh=_sc_mesh(),
        compiler_params=pltpu.CompilerParams(needs_layout_passes=False),
        scratch_types=[
            pltpu.VMEM((GB,), jnp.int32),
            pltpu.VMEM((CAP,), jnp.int32),
            pltpu.VMEM((FB,), jnp.int32),
            pltpu.VMEM((FB,), jnp.int32),
            pltpu.VMEM((FB, D), jnp.float32),
            pltpu.VMEM((AROWS, D), jnp.float32),
        ],
    )
    def sums_k(x_hbm, src_hbm, g_hbm, zeros_hbm, out_hbm,
               gbuf, eid_v, sidx_v, gsel_v, rows_v, acc_v):
        cid = lax.axis_index("c")
        sid = lax.axis_index("s")
        w = cid * NS + sid
        iota = _iota16()
        sentinel = EPAD - 1

        def flush(off, base):
            nb = (off + LANES + FB - 1) // FB
            for k in range(5):
                idx = off + k * LANES + iota
                plsc.store_scatter(eid_v, [idx],
                                   jnp.full((LANES,), sentinel, jnp.int32),
                                   mask=idx < nb * FB)

            def batch(b, carry):
                esl = eid_v.at[pl.ds(b * FB, FB)]
                pltpu.sync_copy(src_hbm.at[esl], sidx_v)
                pltpu.sync_copy(g_hbm.at[esl], gsel_v)
                pltpu.sync_copy(x_hbm.at[sidx_v], rows_v)
                for sub in range(FB // LANES):
                    g16 = gsel_v[pl.ds(sub * LANES, LANES)]
                    valid = (g16 >= base) & (g16 < base + ACC_ROWS)
                    local = jnp.where(valid, g16 - base, ACC_ROWS)
                    rbase = sub * LANES + iota

                    def feat(j, c2):
                        j16 = jnp.full((LANES,), 0, jnp.int32) + j
                        col = plsc.load_gather(rows_v, [rbase, j16])
                        plsc.addupdate_scatter(acc_v, [local, j16], col,
                                               mask=valid)
                        return c2

                    lax.fori_loop(0, D, feat, 0)
                return carry

            lax.fori_loop(0, nb, batch, 0)

        def pass_body(p, carry):
            base = (p * NW + w) * ACC_ROWS
            pltpu.sync_copy(zeros_hbm, acc_v)

            def sb_body(sb, carry2):
                pltpu.sync_copy(g_hbm.at[pl.ds(sb * GB, GB)], gbuf)

                def scan(i, off):
                    g16 = gbuf[pl.ds(i * LANES, LANES)]
                    m = (g16 >= base) & (g16 < base + ACC_ROWS)
                    mi = m.astype(jnp.int32)
                    pos = off + plsc.cumsum(mi) - mi
                    eid = sb * GB + i * LANES + iota
                    plsc.store_scatter(eid_v, [pos], eid, mask=m)
                    return off + jnp.sum(mi)

                off = lax.fori_loop(0, GB // LANES, scan, 0)
                flush(off, base)
                return carry2

            lax.fori_loop(0, NSB, sb_body, 0)
            pltpu.sync_copy(acc_v.at[pl.ds(0, ACC_ROWS)],
                            out_hbm.at[pl.ds(base, ACC_ROWS)])
            return carry

        lax.fori_loop(0, PASSES, pass_body, 0)

    return sums_k


def _make_tail_gather(N, TAIL, Q, H):
    per = Q // NW

    @functools.partial(
        pl.kernel,
        out_type=jax.ShapeDtypeStruct((Q, H), jnp.float32),
        mesh=_sc_mesh(),
        compiler_params=pltpu.CompilerParams(needs_layout_passes=False),
        scratch_types=[
            pltpu.VMEM((per,), jnp.int32),
            pltpu.VMEM((per,), jnp.int32),
            pltpu.VMEM((per, H), jnp.float32),
        ],
    )
    def gather_k(h_hbm, idx_hbm, out_hbm, idx_v, base_v, rows_v):
        cid = lax.axis_index("c")
        sid = lax.axis_index("s")
        w = cid * NS + sid
        pltpu.sync_copy(idx_hbm.at[pl.ds(w * per, per)], idx_v)

        def body(i, carry):
            sl = pl.ds(i * LANES, LANES)
            base_v[sl] = idx_v[sl] + (N - TAIL)
            return carry

        lax.fori_loop(0, per // LANES, body, 0)
        pltpu.sync_copy(h_hbm.at[base_v], rows_v)
        pltpu.sync_copy(rows_v, out_hbm.at[pl.ds(w * per, per)])

    return gather_k


def _tc_wcat(comp_p, basis_flat, csplit):
    Rr, NBp = comp_p.shape
    K = basis_flat.shape[1]
    BK = K // csplit

    def k(c_ref, b_ref, o_ref):
        o_ref[...] = jnp.dot(c_ref[...], b_ref[...],
                             preferred_element_type=jnp.float32)

    return pl.pallas_call(
        k,
        grid=(csplit,),
        in_specs=[
            pl.BlockSpec((Rr, NBp), lambda i: (0, 0)),
            pl.BlockSpec((NBp, BK), lambda i: (0, i)),
        ],
        out_specs=pl.BlockSpec((Rr, BK), lambda i: (0, i)),
        out_shape=jax.ShapeDtypeStruct((Rr, K), jnp.float32),
    )(comp_p, basis_flat)


def _tc_layer(x, sums3, cnt, root, bias, w3, BM):
    Np, Din = x.shape
    R = w3.shape[0]
    D = w3.shape[1]
    H = w3.shape[2]
    CP = cnt.shape[1]
    grid = Np // BM

    def k(x_ref, s_ref, c_ref, root_ref, bias_ref, w_ref, o_ref):
        acc = jnp.dot(x_ref[...], root_ref[...],
                      preferred_element_type=jnp.float32)
        for r in range(R):
            inv = 1.0 / jnp.maximum(c_ref[:, r:r + 1], 1.0)
            agg = s_ref[:, r, :] * inv
            acc = acc + jnp.dot(agg, w_ref[r],
                                preferred_element_type=jnp.float32)
        o_ref[...] = acc + bias_ref[...]

    return pl.pallas_call(
        k,
        grid=(grid,),
        in_specs=[
            pl.BlockSpec((BM, Din), lambda i: (i, 0)),
            pl.BlockSpec((BM, R, D), lambda i: (i, 0, 0)),
            pl.BlockSpec((BM, CP), lambda i: (i, 0)),
            pl.BlockSpec((Din, H), lambda i: (0, 0)),
            pl.BlockSpec((1, H), lambda i: (0, 0)),
            pl.BlockSpec((R, D, H), lambda i: (0, 0, 0)),
        ],
        out_specs=pl.BlockSpec((BM, H), lambda i: (i, 0)),
        out_shape=jax.ShapeDtypeStruct((Np, H), jnp.float32),
    )(x, sums3, cnt, root, bias.reshape(1, H), w3)


def _acc_rows(D):
    return (245 * 1024 // (D * 4) - 8) // 8 * 8


def kernel(x, edge_index, edge_type, delIndexes, basis1, comp1, root1, bias1,
           basis2, comp2, root2, bias2):
    N, IN = x.shape
    HID = root1.shape[1]
    OUT = root2.shape[1]
    R = comp1.shape[0]
    NB = comp1.shape[1]
    E = edge_type.shape[0]
    Q = delIndexes.shape[0]
    TAIL = 1000

    EPT = -(-E // NW)
    EPT = -(-EPT // 128) * 128
    EPAD = EPT * NW
    pad = EPAD - E
    src = jnp.concatenate(
        [edge_index[0], jnp.zeros((pad,), jnp.int32)]).astype(jnp.int32)
    dst = jnp.concatenate(
        [edge_index[1], jnp.full((pad,), N, jnp.int32)]).astype(jnp.int32)
    typ = jnp.concatenate(
        [edge_type, jnp.zeros((pad,), jnp.int32)]).astype(jnp.int32)

    SLOTS = -(-((N + 1) * R) // (NW * 128)) * (NW * 128)
    cnt_parts, g = _make_slots_kernel(EPT, SLOTS, R)(dst, typ)
    cnt = _make_cnt_reduce(SLOTS)(cnt_parts)
    cnt = cnt[:N * R].reshape(N, R)
    CP = 128
    cnt = jnp.pad(cnt, ((0, 0), (0, CP - R)))

    AR1 = _acc_rows(IN)
    P1 = -(-SLOTS // (NW * AR1))
    zeros1 = jnp.zeros((AR1 + 8, IN), jnp.float32)
    sums1 = _make_sums(EPAD, IN, AR1, P1)(x, src, g, zeros1)
    sums1 = sums1[:N * R].reshape(N, R, IN)
    w1 = _tc_wcat(jnp.pad(comp1, ((0, 0), (0, 2))),
                  jnp.pad(basis1.reshape(NB, IN * HID), ((0, 2), (0, 0))),
                  8)
    w1 = w1.reshape(R, IN, HID)
    h1 = _tc_layer(x, sums1, cnt, root1, bias1, w1, 1000)

    AR2 = _acc_rows(HID)
    P2 = -(-SLOTS // (NW * AR2))
    zeros2 = jnp.zeros((AR2 + 8, HID), jnp.float32)
    sums2 = _make_sums(EPAD, HID, AR2, P2)(h1, src, g, zeros2)
    sums2 = sums2[:N * R].reshape(N, R, HID)
    w2 = _tc_wcat(jnp.pad(comp2, ((0, 0), (0, 2))),
                  jnp.pad(basis2.reshape(NB, HID * OUT), ((0, 2), (0, 0))),
                  8)
    w2 = w2.reshape(R, HID, OUT)
    h2 = _tc_layer(h1, sums2, cnt, root2, bias2, w2, 400)

    out = _make_tail_gather(N, TAIL, Q, OUT)(h2, delIndexes.astype(jnp.int32))
    return out
